# baseline (speedup 1.0000x reference)
import functools

import jax
import jax.numpy as jnp
from jax.experimental import pallas as pl
from jax.experimental.pallas import tpu as pltpu

_B = 5000
_P = 32
_C = 128
_HID = 64
_NS = 100
_NBP = 256
_NBF = 512
_GP = (_B + _NBP - 1) // _NBP
_GF = (_B + _NBF - 1) // _NBF
_SL, _LN = 8, 640
_BPAD = _SL * _LN


def _body(xa_ref, xc_ref, w1_ref, b1_ref, w2_ref, b2_ref, w3_ref, b3_ref,
          w4_ref, b4_ref, wfc_ref, bfc_ref, o_ref,
          y_s, yt_s, ahub_s, anon_s, col_s):
    i = pl.program_id(0)

    @pl.when(i < _GP)
    def _pool():
        yb = jnp.max(xa_ref[...], axis=1)
        rows = jax.lax.broadcasted_iota(jnp.int32, (_NBP, 1), 0) + i * _NBP
        ybs = jnp.where(rows < _B, yb, 0.0)
        y_s[pl.ds(i * _NBP, _NBP), :] = ybs
        yt_s[:, pl.ds(pl.multiple_of(i * _NBP, _NBP), _NBP)] = ybs.T

    @pl.when(i == _GP)
    def _fps_chain():
        yt = yt_s[...]
        y3 = jnp.stack([yt[:, s * _LN:(s + 1) * _LN] for s in range(_SL)],
                       axis=1)
        node_id = (jax.lax.broadcasted_iota(jnp.int32, (_SL, _LN), 0) * _LN
                   + jax.lax.broadcasted_iota(jnp.int32, (_SL, _LN), 1))
        lane_c = jax.lax.broadcasted_iota(jnp.int32, (1, 128), 1)

        dist0 = jnp.where(node_id < _B, jnp.inf, -jnp.inf).astype(jnp.float32)
        col0 = jnp.where(lane_c == 0, 0, -1)

        def body(it, carry):
            dist, colv, last, dcnt = carry
            ylast = y_s[pl.ds(last, 1), :]
            yl3 = ylast.reshape(_C, 1, 1)
            acc = (y3 - yl3) ** 2
            for h in (64, 32, 16, 8, 4, 2, 1):
                acc = acc[:h] + acc[h:2 * h]
            d = acc.reshape(_SL, _LN)
            dist = jnp.minimum(dist, d)
            m = jnp.max(dist)
            nxt = jnp.min(jnp.where(dist == m, node_id, _BPAD)).astype(jnp.int32)
            colv = jnp.where(lane_c == it, nxt, colv)
            return dist, colv, nxt, dcnt + (m > 0.0).astype(jnp.float32)

        carry = (dist0, col0, jnp.int32(0), jnp.float32(1))
        _, colv, _, dcnt = jax.lax.fori_loop(1, _NS, body, carry)
        col_s[...] = colv

        nf = jnp.float32(_B)
        dn = dcnt
        mean_y = jnp.sum(y_s[...], axis=0, keepdims=True) / nf

        def feast(mz, w_ref, b_ref):
            h = jnp.dot(mz, w_ref[...],
                        preferred_element_type=jnp.float32) + b_ref[...]
            return h, b_ref[...]

        h1, n1 = feast(mean_y, w1_ref, b1_ref)
        h1, n1 = jax.nn.relu(h1), jax.nn.relu(n1)
        m1 = (dn * h1 + (nf - dn) * n1) / nf
        h2, n2 = feast(m1, w2_ref, b2_ref)
        h2, n2 = jax.nn.relu(h2), jax.nn.relu(n2)
        m2 = (dn * h2 + (nf - dn) * n2) / nf
        h3, n3 = feast(m2, w3_ref, b3_ref)
        h3, n3 = jax.nn.relu(h3), jax.nn.relu(n3)
        m3 = (dn * h3 + (nf - dn) * n3) / nf
        vh, vn = feast(m3, w4_ref, b4_ref)

        mm = jnp.maximum(vh, vn)
        eh = jnp.exp(vh - mm)
        en = jnp.exp(vn - mm)
        z = dn * eh + (nf - dn) * en
        ahub_s[...] = eh / z
        anon_s[...] = en / z

    @pl.when(i > _GP)
    def _final():
        j = i - _GP - 1
        colv = col_s[...]
        rows = jax.lax.broadcasted_iota(jnp.int32, (_NBF, 1), 0) + j * _NBF
        hub = jnp.max((rows == colv).astype(jnp.float32), axis=1, keepdims=True)
        ah = ahub_s[...]
        an = anon_s[...]
        att = an + hub * (ah - an)

        xb = xc_ref[...]
        x2 = xb.reshape(_NBF * _P, _C)
        attr = jnp.broadcast_to(att[:, None, :],
                                (_NBF, _P, _C)).reshape(_NBF * _P, _C)
        wtop = wfc_ref[0:_C, :]
        wbot = wfc_ref[_C:2 * _C, :]
        out = (jnp.dot(x2 * attr, wtop, preferred_element_type=jnp.float32)
               + jnp.dot(x2, wbot, preferred_element_type=jnp.float32)
               + bfc_ref[...])
        o_ref[...] = out.reshape(_NBF, _P, _C)


def kernel(x, W1, u1, c1, b1, W2, u2, c2, b2, W3, u3, c3, b3, W4, u4, c4, b4, Wfc, bfc):
    f32 = jnp.float32

    def full(shape):
        return pl.BlockSpec(shape, lambda *_: tuple(0 for _ in shape))

    out = pl.pallas_call(
        _body,
        grid=(_GP + 1 + _GF,),
        in_specs=[
            pl.BlockSpec((_NBP, _P, _C),
                         lambda i: (jnp.minimum(i, _GP - 1), 0, 0)),
            pl.BlockSpec((_NBF, _P, _C),
                         lambda i: (jnp.clip(i - _GP - 1, 0, _GF - 1), 0, 0)),
            full((_C, _C)),
            full((1, _C)),
            full((_C, _HID)),
            full((1, _HID)),
            full((_HID, _C)),
            full((1, _C)),
            full((_C, _C)),
            full((1, _C)),
            full((2 * _C, _C)),
            full((1, _C)),
        ],
        out_specs=pl.BlockSpec((_NBF, _P, _C),
                               lambda i: (jnp.clip(i - _GP - 1, 0, _GF - 1),
                                          0, 0)),
        out_shape=jax.ShapeDtypeStruct((_B, _P, _C), f32),
        scratch_shapes=[
            pltpu.VMEM((_BPAD, _C), f32),
            pltpu.VMEM((_C, _BPAD), f32),
            pltpu.VMEM((1, _C), f32),
            pltpu.VMEM((1, _C), f32),
            pltpu.VMEM((1, 128), jnp.int32),
        ],
        compiler_params=pltpu.CompilerParams(
            dimension_semantics=("arbitrary",)),
    )(x, x, W1, b1.reshape(1, _C), W2, b2.reshape(1, _HID),
      W3, b3.reshape(1, _C), W4, b4.reshape(1, _C), Wfc, bfc.reshape(1, _C))

    return out


# speedup vs baseline: 1.1979x; 1.1979x over previous
import functools

import jax
import jax.numpy as jnp
from jax.experimental import pallas as pl
from jax.experimental.pallas import tpu as pltpu

_B = 5000
_P = 32
_C = 128
_HID = 64
_NS = 100
_NBP = 256
_NBF = 512
_GP = (_B + _NBP - 1) // _NBP
_GF = (_B + _NBF - 1) // _NBF
_SL, _LN = 8, 640
_BPAD = _SL * _LN


def _body(xa_ref, xc_ref, w1_ref, b1_ref, w2_ref, b2_ref, w3_ref, b3_ref,
          w4_ref, b4_ref, wfc_ref, bfc_ref, o_ref,
          y_s, yt_s, ahub_s, anon_s, col_s):
    i = pl.program_id(0)

    @pl.when(i < _GP)
    def _pool():
        yb = jnp.max(xa_ref[...], axis=1)
        rows = jax.lax.broadcasted_iota(jnp.int32, (_NBP, 1), 0) + i * _NBP
        ybs = jnp.where(rows < _B, yb, 0.0)
        y_s[pl.ds(i * _NBP, _NBP), :] = ybs
        yt_s[:, pl.ds(pl.multiple_of(i * _NBP, _NBP), _NBP)] = ybs.T

    @pl.when(i == _GP)
    def _fps_chain():
        yt = yt_s[...]
        y3 = jnp.stack([yt[:, s * _LN:(s + 1) * _LN] for s in range(_SL)],
                       axis=1)
        node_id = (jax.lax.broadcasted_iota(jnp.int32, (_SL, _LN), 0) * _LN
                   + jax.lax.broadcasted_iota(jnp.int32, (_SL, _LN), 1))
        lane_c = jax.lax.broadcasted_iota(jnp.int32, (1, 128), 1)

        dist0 = jnp.where(node_id < _B, jnp.inf, -jnp.inf).astype(jnp.float32)
        col0 = jnp.where(lane_c == 0, 0, -1)

        def body(it, carry):
            dist, colv, last, dcnt = carry
            ylast = y_s[pl.ds(last, 1), :]
            yl3 = ylast.reshape(_C, 1, 1)
            d = jnp.sum((y3 - yl3) ** 2, axis=0)
            dist = jnp.minimum(dist, d)
            m = jnp.max(dist)
            nxt = jnp.min(jnp.where(dist == m, node_id, _BPAD)).astype(jnp.int32)
            colv = jnp.where(lane_c == it, nxt, colv)
            return dist, colv, nxt, dcnt + (m > 0.0).astype(jnp.float32)

        carry = (dist0, col0, jnp.int32(0), jnp.float32(1))
        _, colv, _, dcnt = jax.lax.fori_loop(1, _NS, body, carry)
        col_s[...] = colv

        nf = jnp.float32(_B)
        dn = dcnt
        mean_y = jnp.sum(y_s[...], axis=0, keepdims=True) / nf

        def feast(mz, w_ref, b_ref):
            h = jnp.dot(mz, w_ref[...],
                        preferred_element_type=jnp.float32) + b_ref[...]
            return h, b_ref[...]

        h1, n1 = feast(mean_y, w1_ref, b1_ref)
        h1, n1 = jax.nn.relu(h1), jax.nn.relu(n1)
        m1 = (dn * h1 + (nf - dn) * n1) / nf
        h2, n2 = feast(m1, w2_ref, b2_ref)
        h2, n2 = jax.nn.relu(h2), jax.nn.relu(n2)
        m2 = (dn * h2 + (nf - dn) * n2) / nf
        h3, n3 = feast(m2, w3_ref, b3_ref)
        h3, n3 = jax.nn.relu(h3), jax.nn.relu(n3)
        m3 = (dn * h3 + (nf - dn) * n3) / nf
        vh, vn = feast(m3, w4_ref, b4_ref)

        mm = jnp.maximum(vh, vn)
        eh = jnp.exp(vh - mm)
        en = jnp.exp(vn - mm)
        z = dn * eh + (nf - dn) * en
        ahub_s[...] = eh / z
        anon_s[...] = en / z

    @pl.when(i > _GP)
    def _final():
        j = i - _GP - 1
        colv = col_s[...]
        rows = jax.lax.broadcasted_iota(jnp.int32, (_NBF, 1), 0) + j * _NBF
        hub = jnp.max((rows == colv).astype(jnp.float32), axis=1, keepdims=True)
        ah = ahub_s[...]
        an = anon_s[...]
        att = an + hub * (ah - an)

        xb = xc_ref[...]
        x2 = xb.reshape(_NBF * _P, _C)
        attr = jnp.broadcast_to(att[:, None, :],
                                (_NBF, _P, _C)).reshape(_NBF * _P, _C)
        wtop = wfc_ref[0:_C, :]
        wbot = wfc_ref[_C:2 * _C, :]
        out = (jnp.dot(x2 * attr, wtop, preferred_element_type=jnp.float32)
               + jnp.dot(x2, wbot, preferred_element_type=jnp.float32)
               + bfc_ref[...])
        o_ref[...] = out.reshape(_NBF, _P, _C)


def kernel(x, W1, u1, c1, b1, W2, u2, c2, b2, W3, u3, c3, b3, W4, u4, c4, b4, Wfc, bfc):
    f32 = jnp.float32

    def full(shape):
        return pl.BlockSpec(shape, lambda *_: tuple(0 for _ in shape))

    out = pl.pallas_call(
        _body,
        grid=(_GP + 1 + _GF,),
        in_specs=[
            pl.BlockSpec((_NBP, _P, _C),
                         lambda i: (jnp.minimum(i, _GP - 1), 0, 0)),
            pl.BlockSpec((_NBF, _P, _C),
                         lambda i: (jnp.clip(i - _GP - 1, 0, _GF - 1), 0, 0)),
            full((_C, _C)),
            full((1, _C)),
            full((_C, _HID)),
            full((1, _HID)),
            full((_HID, _C)),
            full((1, _C)),
            full((_C, _C)),
            full((1, _C)),
            full((2 * _C, _C)),
            full((1, _C)),
        ],
        out_specs=pl.BlockSpec((_NBF, _P, _C),
                               lambda i: (jnp.clip(i - _GP - 1, 0, _GF - 1),
                                          0, 0)),
        out_shape=jax.ShapeDtypeStruct((_B, _P, _C), f32),
        scratch_shapes=[
            pltpu.VMEM((_BPAD, _C), f32),
            pltpu.VMEM((_C, _BPAD), f32),
            pltpu.VMEM((1, _C), f32),
            pltpu.VMEM((1, _C), f32),
            pltpu.VMEM((1, 128), jnp.int32),
        ],
        compiler_params=pltpu.CompilerParams(
            dimension_semantics=("arbitrary",)),
    )(x, x, W1, b1.reshape(1, _C), W2, b2.reshape(1, _HID),
      W3, b3.reshape(1, _C), W4, b4.reshape(1, _C), Wfc, bfc.reshape(1, _C))

    return out


# speedup vs baseline: 1.2550x; 1.0476x over previous
import functools

import jax
import jax.numpy as jnp
from jax.experimental import pallas as pl
from jax.experimental.pallas import tpu as pltpu

_B = 5000
_P = 32
_C = 128
_HID = 64
_NS = 100
_NBP = 256
_NBF = 512
_GP = (_B + _NBP - 1) // _NBP
_GF = (_B + _NBF - 1) // _NBF
_SL, _LN = 8, 640
_BPAD = _SL * _LN


def _body(xa_ref, xc_ref, w1_ref, b1_ref, w2_ref, b2_ref, w3_ref, b3_ref,
          w4_ref, b4_ref, wfc_ref, bfc_ref, o_ref,
          y_s, yt_s, ahub_s, anon_s, col_s):
    i = pl.program_id(0)

    @pl.when(i < _GP)
    def _pool():
        yb = jnp.max(xa_ref[...], axis=1)
        rows = jax.lax.broadcasted_iota(jnp.int32, (_NBP, 1), 0) + i * _NBP
        ybs = jnp.where(rows < _B, yb, 0.0)
        y_s[pl.ds(i * _NBP, _NBP), :] = ybs
        yt_s[:, pl.ds(pl.multiple_of(i * _NBP, _NBP), _NBP)] = ybs.T

    @pl.when(i == _GP)
    def _fps_chain():
        yt = yt_s[...]
        y3 = jnp.stack([yt[:, s * _LN:(s + 1) * _LN] for s in range(_SL)],
                       axis=1)
        node_id = (jax.lax.broadcasted_iota(jnp.int32, (_SL, _LN), 0) * _LN
                   + jax.lax.broadcasted_iota(jnp.int32, (_SL, _LN), 1))
        lane_c = jax.lax.broadcasted_iota(jnp.int32, (1, 128), 1)

        dist0 = jnp.where(node_id < _B, jnp.inf, -jnp.inf).astype(jnp.float32)
        col0 = jnp.where(lane_c == 0, 0, -1)
        ynorm = jnp.sum(y3 * y3, axis=0)

        def body(it, carry):
            dist, colv, last, dcnt = carry
            ylast = y_s[pl.ds(last, 1), :]
            yl3 = ylast.reshape(_C, 1, 1)
            c0 = jnp.sum(ylast * ylast)
            d = (ynorm + c0) - 2.0 * jnp.sum(y3 * yl3, axis=0)
            dist = jnp.minimum(dist, d)
            m = jnp.max(dist)
            nxt = jnp.min(jnp.where(dist == m, node_id, _BPAD)).astype(jnp.int32)
            colv = jnp.where(lane_c == it, nxt, colv)
            return dist, colv, nxt, dcnt + (m > 0.0).astype(jnp.float32)

        carry = (dist0, col0, jnp.int32(0), jnp.float32(1))
        _, colv, _, dcnt = jax.lax.fori_loop(1, _NS, body, carry)
        col_s[...] = colv

        nf = jnp.float32(_B)
        dn = dcnt
        mean_y = jnp.sum(y_s[...], axis=0, keepdims=True) / nf

        def feast(mz, w_ref, b_ref):
            h = jnp.dot(mz, w_ref[...],
                        preferred_element_type=jnp.float32) + b_ref[...]
            return h, b_ref[...]

        h1, n1 = feast(mean_y, w1_ref, b1_ref)
        h1, n1 = jax.nn.relu(h1), jax.nn.relu(n1)
        m1 = (dn * h1 + (nf - dn) * n1) / nf
        h2, n2 = feast(m1, w2_ref, b2_ref)
        h2, n2 = jax.nn.relu(h2), jax.nn.relu(n2)
        m2 = (dn * h2 + (nf - dn) * n2) / nf
        h3, n3 = feast(m2, w3_ref, b3_ref)
        h3, n3 = jax.nn.relu(h3), jax.nn.relu(n3)
        m3 = (dn * h3 + (nf - dn) * n3) / nf
        vh, vn = feast(m3, w4_ref, b4_ref)

        mm = jnp.maximum(vh, vn)
        eh = jnp.exp(vh - mm)
        en = jnp.exp(vn - mm)
        z = dn * eh + (nf - dn) * en
        ahub_s[...] = eh / z
        anon_s[...] = en / z

    @pl.when(i > _GP)
    def _final():
        j = i - _GP - 1
        colv = col_s[...]
        rows = jax.lax.broadcasted_iota(jnp.int32, (_NBF, 1), 0) + j * _NBF
        hub = jnp.max((rows == colv).astype(jnp.float32), axis=1, keepdims=True)
        ah = ahub_s[...]
        an = anon_s[...]
        att = an + hub * (ah - an)

        xb = xc_ref[...]
        x2 = xb.reshape(_NBF * _P, _C)
        attr = jnp.broadcast_to(att[:, None, :],
                                (_NBF, _P, _C)).reshape(_NBF * _P, _C)
        wtop = wfc_ref[0:_C, :]
        wbot = wfc_ref[_C:2 * _C, :]
        out = (jnp.dot(x2 * attr, wtop, preferred_element_type=jnp.float32)
               + jnp.dot(x2, wbot, preferred_element_type=jnp.float32)
               + bfc_ref[...])
        o_ref[...] = out.reshape(_NBF, _P, _C)


def kernel(x, W1, u1, c1, b1, W2, u2, c2, b2, W3, u3, c3, b3, W4, u4, c4, b4, Wfc, bfc):
    f32 = jnp.float32

    def full(shape):
        return pl.BlockSpec(shape, lambda *_: tuple(0 for _ in shape))

    out = pl.pallas_call(
        _body,
        grid=(_GP + 1 + _GF,),
        in_specs=[
            pl.BlockSpec((_NBP, _P, _C),
                         lambda i: (jnp.minimum(i, _GP - 1), 0, 0)),
            pl.BlockSpec((_NBF, _P, _C),
                         lambda i: (jnp.clip(i - _GP - 1, 0, _GF - 1), 0, 0)),
            full((_C, _C)),
            full((1, _C)),
            full((_C, _HID)),
            full((1, _HID)),
            full((_HID, _C)),
            full((1, _C)),
            full((_C, _C)),
            full((1, _C)),
            full((2 * _C, _C)),
            full((1, _C)),
        ],
        out_specs=pl.BlockSpec((_NBF, _P, _C),
                               lambda i: (jnp.clip(i - _GP - 1, 0, _GF - 1),
                                          0, 0)),
        out_shape=jax.ShapeDtypeStruct((_B, _P, _C), f32),
        scratch_shapes=[
            pltpu.VMEM((_BPAD, _C), f32),
            pltpu.VMEM((_C, _BPAD), f32),
            pltpu.VMEM((1, _C), f32),
            pltpu.VMEM((1, _C), f32),
            pltpu.VMEM((1, 128), jnp.int32),
        ],
        compiler_params=pltpu.CompilerParams(
            dimension_semantics=("arbitrary",)),
    )(x, x, W1, b1.reshape(1, _C), W2, b2.reshape(1, _HID),
      W3, b3.reshape(1, _C), W4, b4.reshape(1, _C), Wfc, bfc.reshape(1, _C))

    return out


# speedup vs baseline: 1.2587x; 1.0029x over previous
import jax
import jax.numpy as jnp
from jax.experimental import pallas as pl
from jax.experimental.pallas import tpu as pltpu

_B = 5000
_P = 32
_C = 128
_HID = 64
_NS = 100
_NBP = 256
_NBF = 512
_GP = (_B + _NBP - 1) // _NBP
_GF = (_B + _NBF - 1) // _NBF
_SL, _LN = 8, 640
_BPAD = _SL * _LN


def _body(xa_ref, xc_ref, w1_ref, b1_ref, w2_ref, b2_ref, w3_ref, b3_ref,
          w4_ref, b4_ref, wfc_ref, bfc_ref, o_ref,
          y_s, yt_s, ahub_s, anon_s, col_s):
    i = pl.program_id(0)

    @pl.when(i < _GP)
    def _pool():
        yb = jnp.max(xa_ref[...], axis=1)
        rows = jax.lax.broadcasted_iota(jnp.int32, (_NBP, 1), 0) + i * _NBP
        ybs = jnp.where(rows < _B, yb, 0.0)
        y_s[pl.ds(i * _NBP, _NBP), :] = ybs
        yt_s[:, pl.ds(pl.multiple_of(i * _NBP, _NBP), _NBP)] = ybs.T

    @pl.when(i == _GP)
    def _fps_chain():
        yt = yt_s[...]
        y3 = jnp.stack([yt[:, s * _LN:(s + 1) * _LN] for s in range(_SL)],
                       axis=1)
        node_id = (jax.lax.broadcasted_iota(jnp.int32, (_SL, _LN), 0) * _LN
                   + jax.lax.broadcasted_iota(jnp.int32, (_SL, _LN), 1))
        lane_c = jax.lax.broadcasted_iota(jnp.int32, (1, 128), 1)

        dist0 = jnp.where(node_id < _B, jnp.inf, -jnp.inf).astype(jnp.float32)
        col0 = jnp.where(lane_c == 0, 0, -1)
        ynorm = jnp.sum(y3 * y3, axis=0)

        def body(it, carry):
            dist, colv, last, dcnt = carry
            ylast = y_s[pl.ds(last, 1), :]
            yl3 = ylast.reshape(_C, 1, 1)
            c0 = jnp.sum(ylast * ylast)
            d = (ynorm + c0) - 2.0 * jnp.sum(y3 * yl3, axis=0)
            dist = jnp.minimum(dist, d)
            m = jnp.max(dist)
            nxt = jnp.min(jnp.where(dist == m, node_id, _BPAD)).astype(jnp.int32)
            colv = jnp.where(lane_c == it, nxt, colv)
            return dist, colv, nxt, dcnt + (m > 0.0).astype(jnp.float32)

        carry = (dist0, col0, jnp.int32(0), jnp.float32(1))
        _, colv, _, dcnt = jax.lax.fori_loop(1, _NS, body, carry)
        col_s[...] = colv

        nf = jnp.float32(_B)
        dn = dcnt
        mean_y = jnp.sum(y_s[...], axis=0, keepdims=True) / nf

        def feast(mz, w_ref, b_ref):
            h = jnp.dot(mz, w_ref[...],
                        preferred_element_type=jnp.float32) + b_ref[...]
            return h, b_ref[...]

        h1, n1 = feast(mean_y, w1_ref, b1_ref)
        h1, n1 = jax.nn.relu(h1), jax.nn.relu(n1)
        m1 = (dn * h1 + (nf - dn) * n1) / nf
        h2, n2 = feast(m1, w2_ref, b2_ref)
        h2, n2 = jax.nn.relu(h2), jax.nn.relu(n2)
        m2 = (dn * h2 + (nf - dn) * n2) / nf
        h3, n3 = feast(m2, w3_ref, b3_ref)
        h3, n3 = jax.nn.relu(h3), jax.nn.relu(n3)
        m3 = (dn * h3 + (nf - dn) * n3) / nf
        vh, vn = feast(m3, w4_ref, b4_ref)

        mm = jnp.maximum(vh, vn)
        eh = jnp.exp(vh - mm)
        en = jnp.exp(vn - mm)
        z = dn * eh + (nf - dn) * en
        ahub_s[...] = eh / z
        anon_s[...] = en / z

    @pl.when(i > _GP)
    def _final():
        j = i - _GP - 1
        colv = col_s[...]
        rows = jax.lax.broadcasted_iota(jnp.int32, (_NBF, 1), 0) + j * _NBF
        hub = jnp.max((rows == colv).astype(jnp.float32), axis=1, keepdims=True)
        ah = ahub_s[...]
        an = anon_s[...]
        att = an + hub * (ah - an)

        xb = xc_ref[...]
        x2 = xb.reshape(_NBF * _P, _C)
        attr = jnp.broadcast_to(att[:, None, :],
                                (_NBF, _P, _C)).reshape(_NBF * _P, _C)
        wtop = wfc_ref[0:_C, :]
        wbot = wfc_ref[_C:2 * _C, :]
        out = (jnp.dot(x2 * attr, wtop, preferred_element_type=jnp.float32)
               + jnp.dot(x2, wbot, preferred_element_type=jnp.float32)
               + bfc_ref[...])
        o_ref[...] = out.reshape(_NBF, _P, _C)


def kernel(x, W1, u1, c1, b1, W2, u2, c2, b2, W3, u3, c3, b3, W4, u4, c4, b4, Wfc, bfc):
    f32 = jnp.float32

    def full(shape):
        return pl.BlockSpec(shape, lambda *_: tuple(0 for _ in shape))

    out = pl.pallas_call(
        _body,
        grid=(_GP + 1 + _GF,),
        in_specs=[
            pl.BlockSpec((_NBP, _P, _C),
                         lambda i: (jnp.minimum(i, _GP - 1), 0, 0)),
            pl.BlockSpec((_NBF, _P, _C),
                         lambda i: (jnp.clip(i - _GP - 1, 0, _GF - 1), 0, 0)),
            full((_C, _C)),
            full((1, _C)),
            full((_C, _HID)),
            full((1, _HID)),
            full((_HID, _C)),
            full((1, _C)),
            full((_C, _C)),
            full((1, _C)),
            full((2 * _C, _C)),
            full((1, _C)),
        ],
        out_specs=pl.BlockSpec((_NBF, _P, _C),
                               lambda i: (jnp.clip(i - _GP - 1, 0, _GF - 1),
                                          0, 0)),
        out_shape=jax.ShapeDtypeStruct((_B, _P, _C), f32),
        scratch_shapes=[
            pltpu.VMEM((_BPAD, _C), f32),
            pltpu.VMEM((_C, _BPAD), f32),
            pltpu.VMEM((1, _C), f32),
            pltpu.VMEM((1, _C), f32),
            pltpu.VMEM((1, 128), jnp.int32),
        ],
        compiler_params=pltpu.CompilerParams(
            dimension_semantics=("arbitrary",)),
    )(x, x, W1, b1.reshape(1, _C), W2, b2.reshape(1, _HID),
      W3, b3.reshape(1, _C), W4, b4.reshape(1, _C), Wfc, bfc.reshape(1, _C))

    return out
